# baseline (speedup 1.0000x reference)
import functools

import jax
import jax.numpy as jnp
from jax import lax
from jax.experimental import pallas as pl
from jax.experimental.pallas import tpu as pltpu
from jax.experimental.pallas import tpu_sc as plsc

_VOCAB = 1000000
_D = 64
_B = 4096
_S = 200

try:
    _info = plsc.get_sparse_core_info()
    _NC, _NS = _info.num_cores, _info.num_subcores
except Exception:
    _NC, _NS = 2, 16
_NW = _NC * _NS
_BW = _B // _NW
_TROWS = _VOCAB // 2

_mesh = plsc.VectorSubcoreMesh(core_axis_name="c", subcore_axis_name="s")


@functools.partial(
    pl.kernel,
    mesh=_mesh,
    out_type=(
        jax.ShapeDtypeStruct((_S, _D, _B), jnp.float32),
        jax.ShapeDtypeStruct((_S, _B), jnp.float32),
    ),
    compiler_params=pltpu.CompilerParams(
        use_tc_tiling_on_sc=True, needs_layout_passes=False),
    scratch_types=[
        pltpu.VMEM((_S, _BW), jnp.int32),
        pltpu.VMEM((_S, _BW), jnp.float32),
        pltpu.VMEM((_BW,), jnp.int32),
        pltpu.VMEM((_BW,), jnp.int32),
        pltpu.VMEM((_BW,), jnp.int32),
        pltpu.VMEM((_BW,), jnp.int32),
        pltpu.VMEM((_BW, 128), jnp.float32),
        pltpu.VMEM((_BW, 128), jnp.float32),
        pltpu.VMEM((_D, _BW), jnp.float32),
        pltpu.VMEM((_D, _BW), jnp.float32),
        pltpu.SemaphoreType.DMA,
        pltpu.SemaphoreType.DMA,
        pltpu.SemaphoreType.DMA,
        pltpu.SemaphoreType.DMA,
        pltpu.SemaphoreType.DMA,
    ],
)
def _sc_emb(xt_hbm, tbl_hbm, out_hbm, mask_hbm, xt_all, mk_all, i2a, i2b,
            hfa, hfb, ra, rb, oa, ob, gsa, gsb, osa, osb, msem):
    wid = lax.axis_index("s") * _NC + lax.axis_index("c")
    b0 = wid * _BW
    pltpu.sync_copy(xt_hbm.at[:, pl.ds(b0, _BW)], xt_all)

    def build(s, i2, hf):
        for g in range(_BW // 16):
            tv = xt_all[s, pl.ds(g * 16, 16)]
            i2[pl.ds(g * 16, 16)] = lax.shift_right_logical(tv, 1)
            hf[pl.ds(g * 16, 16)] = lax.shift_left(
                lax.bitwise_and(tv, jnp.int32(1)), 6)

    def fire(i2, rows, sem):
        pltpu.async_copy(tbl_hbm.at[i2], rows, sem)

    def wait_g(i2, rows, sem):
        pltpu.make_async_copy(tbl_hbm.at[i2], rows, sem).wait()

    def compact(hf, rows, ov):
        iota = lax.iota(jnp.int32, 16)

        def gbody(g, carry):
            hvs = [hf[pl.ds(g * 64 + 16 * k, 16)] for k in range(4)]
            rowvs = [g * 64 + 16 * k + iota for k in range(4)]
            for hi in range(_D // 16):
                for lo in range(16):
                    dvec = hi * 16 + lax.bitwise_and(lo + iota, 15)
                    vals = [
                        plsc.load_gather(rows, [rowvs[k], hvs[k] + dvec])
                        for k in range(4)
                    ]
                    for k in range(4):
                        plsc.store_scatter(ov, [dvec, rowvs[k]], vals[k])
            return carry

        lax.fori_loop(0, _BW // 64, gbody, 0)

    def wout(s, ov, sem):
        pltpu.async_copy(ov, out_hbm.at[s, :, pl.ds(b0, _BW)], sem)

    def wait_o(ov, sem):
        pltpu.make_async_copy(ov, out_hbm.at[0, :, pl.ds(b0, _BW)], sem).wait()

    def mbody(s, carry):
        for g in range(_BW // 16):
            tv = xt_all[s, pl.ds(g * 16, 16)]
            mk_all[s, pl.ds(g * 16, 16)] = (tv != 0).astype(jnp.float32)
        return carry

    lax.fori_loop(0, _S, mbody, 0)
    pltpu.async_copy(mk_all, mask_hbm.at[:, pl.ds(b0, _BW)], msem)

    build(0, i2a, hfa)
    fire(i2a, ra, gsa)
    build(1, i2b, hfb)

    def body(t, carry):
        s0 = 2 * t
        fire(i2b, rb, gsb)
        wait_g(i2a, ra, gsa)

        @pl.when(t > 0)
        def _():
            wait_o(oa, osa)

        compact(hfa, ra, oa)
        wout(s0, oa, osa)

        @pl.when(t < _S // 2 - 1)
        def _():
            build(s0 + 2, i2a, hfa)
            fire(i2a, ra, gsa)

        wait_g(i2b, rb, gsb)

        @pl.when(t > 0)
        def _():
            wait_o(ob, osb)

        compact(hfb, rb, ob)
        wout(s0 + 1, ob, osb)

        @pl.when(t < _S // 2 - 1)
        def _():
            build(s0 + 3, i2b, hfb)

        return carry

    lax.fori_loop(0, _S // 2, body, 0)
    wait_o(oa, osa)
    wait_o(ob, osb)
    pltpu.make_async_copy(mk_all, mask_hbm.at[:, pl.ds(b0, _BW)], msem).wait()


def kernel(x, table):
    x = x.astype(jnp.int32)
    xt = x.T
    tbl_pairs = table.reshape(_TROWS, 2 * _D)
    outp, mask_t = _sc_emb(xt, tbl_pairs)
    out_emb = jnp.transpose(outp, (0, 2, 1))
    mask = mask_t.T
    return (out_emb, mask)


# speedup vs baseline: 1.0278x; 1.0278x over previous
import functools

import jax
import jax.numpy as jnp
from jax import lax
from jax.experimental import pallas as pl
from jax.experimental.pallas import tpu as pltpu
from jax.experimental.pallas import tpu_sc as plsc

_VOCAB = 1000000
_D = 64
_B = 4096
_S = 200

try:
    _info = plsc.get_sparse_core_info()
    _NC, _NS = _info.num_cores, _info.num_subcores
except Exception:
    _NC, _NS = 2, 16
_NW = _NC * _NS
_BW = _B // _NW
_TROWS = _VOCAB // 2

_mesh = plsc.VectorSubcoreMesh(core_axis_name="c", subcore_axis_name="s")


@functools.partial(
    pl.kernel,
    mesh=_mesh,
    out_type=(
        jax.ShapeDtypeStruct((_S, _D, _B), jnp.float32),
        jax.ShapeDtypeStruct((_S, _B), jnp.float32),
    ),
    compiler_params=pltpu.CompilerParams(
        use_tc_tiling_on_sc=True, needs_layout_passes=False),
    scratch_types=[
        pltpu.VMEM((_S, _BW), jnp.int32),
        pltpu.VMEM((_S, _BW), jnp.float32),
        pltpu.VMEM((_BW,), jnp.int32),
        pltpu.VMEM((_BW,), jnp.int32),
        pltpu.VMEM((_BW,), jnp.int32),
        pltpu.VMEM((_BW,), jnp.int32),
        pltpu.VMEM((_BW, 128), jnp.float32),
        pltpu.VMEM((_BW, 128), jnp.float32),
        pltpu.VMEM((_D, _BW), jnp.float32),
        pltpu.VMEM((_D, _BW), jnp.float32),
        pltpu.SemaphoreType.DMA,
        pltpu.SemaphoreType.DMA,
        pltpu.SemaphoreType.DMA,
        pltpu.SemaphoreType.DMA,
        pltpu.SemaphoreType.DMA,
    ],
)
def _sc_emb(xt_hbm, tbl_hbm, out_hbm, mask_hbm, xt_all, mk_all, i2a, i2b,
            hfa, hfb, ra, rb, oa, ob, gsa, gsb, osa, osb, msem):
    wid = lax.axis_index("s") * _NC + lax.axis_index("c")
    b0 = wid * _BW
    pltpu.sync_copy(xt_hbm.at[:, pl.ds(b0, _BW)], xt_all)

    def build(s, i2, hf):
        for g in range(_BW // 16):
            tv = xt_all[s, pl.ds(g * 16, 16)]
            i2[pl.ds(g * 16, 16)] = lax.shift_right_logical(tv, 1)
            hf[pl.ds(g * 16, 16)] = lax.shift_left(
                lax.bitwise_and(tv, jnp.int32(1)), 6)

    def fire(i2, rows, sem):
        pltpu.async_copy(tbl_hbm.at[i2], rows, sem)

    def wait_g(i2, rows, sem):
        pltpu.make_async_copy(tbl_hbm.at[i2], rows, sem).wait()

    def compact(hf, rows, ov):
        iota = lax.iota(jnp.int32, 16)

        def gbody(g, carry):
            hv0 = hf[pl.ds(g * 32, 16)]
            hv1 = hf[pl.ds(g * 32 + 16, 16)]
            rowv0 = g * 32 + iota
            rowv1 = g * 32 + 16 + iota
            for hi in range(_D // 16):
                for lo in range(16):
                    dvec = hi * 16 + lax.bitwise_and(lo + iota, 15)
                    vals0 = plsc.load_gather(rows, [rowv0, hv0 + dvec])
                    vals1 = plsc.load_gather(rows, [rowv1, hv1 + dvec])
                    plsc.store_scatter(ov, [dvec, rowv0], vals0)
                    plsc.store_scatter(ov, [dvec, rowv1], vals1)
            return carry

        lax.fori_loop(0, _BW // 32, gbody, 0)

    def wout(s, ov, sem):
        pltpu.async_copy(ov, out_hbm.at[s, :, pl.ds(b0, _BW)], sem)

    def wait_o(ov, sem):
        pltpu.make_async_copy(ov, out_hbm.at[0, :, pl.ds(b0, _BW)], sem).wait()

    def mbody(s, carry):
        for g in range(_BW // 16):
            tv = xt_all[s, pl.ds(g * 16, 16)]
            mk_all[s, pl.ds(g * 16, 16)] = (tv != 0).astype(jnp.float32)
        return carry

    lax.fori_loop(0, _S, mbody, 0)
    pltpu.async_copy(mk_all, mask_hbm.at[:, pl.ds(b0, _BW)], msem)

    build(0, i2a, hfa)
    fire(i2a, ra, gsa)
    build(1, i2b, hfb)

    def body(t, carry):
        s0 = 2 * t
        fire(i2b, rb, gsb)
        wait_g(i2a, ra, gsa)

        @pl.when(t > 0)
        def _():
            wait_o(oa, osa)

        compact(hfa, ra, oa)
        wout(s0, oa, osa)

        @pl.when(t < _S // 2 - 1)
        def _():
            build(s0 + 2, i2a, hfa)
            fire(i2a, ra, gsa)

        wait_g(i2b, rb, gsb)

        @pl.when(t > 0)
        def _():
            wait_o(ob, osb)

        compact(hfb, rb, ob)
        wout(s0 + 1, ob, osb)

        @pl.when(t < _S // 2 - 1)
        def _():
            build(s0 + 3, i2b, hfb)

        return carry

    lax.fori_loop(0, _S // 2, body, 0)
    wait_o(oa, osa)
    wait_o(ob, osb)
    pltpu.make_async_copy(mk_all, mask_hbm.at[:, pl.ds(b0, _BW)], msem).wait()


def kernel(x, table):
    x = x.astype(jnp.int32)
    xt = x.T
    tbl_pairs = table.reshape(_TROWS, 2 * _D)
    outp, mask_t = _sc_emb(xt, tbl_pairs)
    out_emb = jnp.transpose(outp, (0, 2, 1))
    mask = mask_t.T
    return (out_emb, mask)


# speedup vs baseline: 1.3075x; 1.2722x over previous
import functools

import jax
import jax.numpy as jnp
from jax import lax
from jax.experimental import pallas as pl
from jax.experimental.pallas import tpu as pltpu
from jax.experimental.pallas import tpu_sc as plsc

_VOCAB = 1000000
_D = 64
_B = 4096
_S = 200

try:
    _info = plsc.get_sparse_core_info()
    _NC, _NS = _info.num_cores, _info.num_subcores
except Exception:
    _NC, _NS = 2, 16
_NW = _NC * _NS
_BW = _B // _NW
_TROWS = _VOCAB // 2

_mesh = plsc.VectorSubcoreMesh(core_axis_name="c", subcore_axis_name="s")


@functools.partial(
    pl.kernel,
    mesh=_mesh,
    out_type=(
        jax.ShapeDtypeStruct((_S, _D, _B), jnp.float32),
        jax.ShapeDtypeStruct((_S, _B), jnp.float32),
    ),
    compiler_params=pltpu.CompilerParams(
        use_tc_tiling_on_sc=True, needs_layout_passes=False),
    scratch_types=[
        pltpu.VMEM((_S, _BW), jnp.int32),
        pltpu.VMEM((_S, _BW), jnp.float32),
        pltpu.VMEM((_BW,), jnp.int32),
        pltpu.VMEM((_BW,), jnp.int32),
        pltpu.VMEM((_BW,), jnp.int32),
        pltpu.VMEM((_BW,), jnp.int32),
        pltpu.VMEM((_BW, 128), jnp.float32),
        pltpu.VMEM((_BW, 128), jnp.float32),
        pltpu.VMEM((_D, _BW), jnp.float32),
        pltpu.VMEM((_D, _BW), jnp.float32),
        pltpu.SemaphoreType.DMA,
        pltpu.SemaphoreType.DMA,
        pltpu.SemaphoreType.DMA,
        pltpu.SemaphoreType.DMA,
        pltpu.SemaphoreType.DMA,
    ],
)
def _sc_emb(xt_hbm, tbl_hbm, out_hbm, mask_hbm, xt_all, mk_all, i2a, i2b,
            hfa, hfb, ra, rb, oa, ob, gsa, gsb, osa, osb, msem):
    wid = lax.axis_index("s") * _NC + lax.axis_index("c")
    b0 = wid * _BW
    pltpu.sync_copy(xt_hbm.at[:, pl.ds(b0, _BW)], xt_all)

    def build(s, i2, hf):
        for g in range(_BW // 16):
            tv = xt_all[s, pl.ds(g * 16, 16)]
            i2[pl.ds(g * 16, 16)] = lax.shift_right_logical(tv, 1)
            hf[pl.ds(g * 16, 16)] = lax.shift_left(
                lax.bitwise_and(tv, jnp.int32(1)), 6)

    def fire(i2, rows, sem):
        pltpu.async_copy(tbl_hbm.at[i2], rows, sem)

    def wait_g(i2, rows, sem):
        pltpu.make_async_copy(tbl_hbm.at[i2], rows, sem).wait()

    def compact(hf, rows, ov):
        iota = lax.iota(jnp.int32, 16)

        def gbody(g, carry):
            hv0 = hf[pl.ds(g * 32, 16)]
            hv1 = hf[pl.ds(g * 32 + 16, 16)]
            rowv0 = g * 32 + iota
            rowv1 = g * 32 + 16 + iota
            for hi in range(_D // 16):
                for lo in range(16):
                    dvec = hi * 16 + lax.bitwise_and(lo + iota, 15)
                    vals0 = plsc.load_gather(rows, [rowv0, hv0 + dvec])
                    vals1 = plsc.load_gather(rows, [rowv1, hv1 + dvec])
                    plsc.store_scatter(ov, [dvec, rowv0], vals0)
                    plsc.store_scatter(ov, [dvec, rowv1], vals1)
            return carry

        lax.fori_loop(0, _BW // 32, gbody, 0)

    def wout(s, ov, sem):
        pltpu.async_copy(ov, out_hbm.at[s, :, pl.ds(b0, _BW)], sem)

    def wait_o(ov, sem):
        pltpu.make_async_copy(ov, out_hbm.at[0, :, pl.ds(b0, _BW)], sem).wait()

    def mbody(s, carry):
        for g in range(_BW // 16):
            tv = xt_all[s, pl.ds(g * 16, 16)]
            mk_all[s, pl.ds(g * 16, 16)] = (tv != 0).astype(jnp.float32)
        return carry

    lax.fori_loop(0, _S, mbody, 0)
    pltpu.async_copy(mk_all, mask_hbm.at[:, pl.ds(b0, _BW)], msem)

    build(0, i2a, hfa)
    fire(i2a, ra, gsa)
    build(1, i2b, hfb)

    def body(t, carry):
        s0 = 2 * t
        fire(i2b, rb, gsb)
        wait_g(i2a, ra, gsa)

        @pl.when(t > 0)
        def _():
            wait_o(oa, osa)

        compact(hfa, ra, oa)
        wout(s0, oa, osa)

        @pl.when(t < _S // 2 - 1)
        def _():
            build(s0 + 2, i2a, hfa)
            fire(i2a, ra, gsa)

        wait_g(i2b, rb, gsb)

        @pl.when(t > 0)
        def _():
            wait_o(ob, osb)

        compact(hfb, rb, ob)
        wout(s0 + 1, ob, osb)

        @pl.when(t < _S // 2 - 1)
        def _():
            build(s0 + 3, i2b, hfb)

        return carry

    lax.fori_loop(0, _S // 2, body, 0)
    wait_o(oa, osa)
    wait_o(ob, osb)
    pltpu.make_async_copy(mk_all, mask_hbm.at[:, pl.ds(b0, _BW)], msem).wait()


_VB = 256
_NVB = _VOCAB // _VB
_VTAIL = _VOCAB - _NVB * _VB


@functools.partial(
    pl.kernel,
    mesh=_mesh,
    out_type=jax.ShapeDtypeStruct((_TROWS, 2 * _D), jnp.float32),
    compiler_params=pltpu.CompilerParams(
        use_tc_tiling_on_sc=True, needs_layout_passes=False),
    scratch_types=[
        pltpu.VMEM((_D, _VB), jnp.float32),
        pltpu.VMEM((_D, _VB), jnp.float32),
        pltpu.VMEM((_VB // 2, 2 * _D), jnp.float32),
        pltpu.VMEM((_VB // 2, 2 * _D), jnp.float32),
        pltpu.VMEM((_D, _VTAIL), jnp.float32),
        pltpu.VMEM((_VTAIL // 2, 2 * _D), jnp.float32),
        pltpu.SemaphoreType.DMA,
        pltpu.SemaphoreType.DMA,
        pltpu.SemaphoreType.DMA,
        pltpu.SemaphoreType.DMA,
    ],
)
def _sc_repack(tt_hbm, out_hbm, ia, ib, pa, pb, tin, tout,
               isa, isb, osa, osb):
    wid = lax.axis_index("s") * _NC + lax.axis_index("c")
    nblk = (_NVB - wid + _NW - 1) // _NW
    iota = lax.iota(jnp.int32, 16)

    def fire(b, slab, sem):
        pltpu.async_copy(tt_hbm.at[:, pl.ds(b * _VB, _VB)], slab, sem)

    def wait_i(slab, sem):
        pltpu.make_async_copy(tt_hbm.at[:, pl.ds(0, _VB)], slab, sem).wait()

    def xpose(slab, pair):
        def vbody(vt, carry):
            vvec0 = vt * 32 + iota
            vvec1 = vt * 32 + 16 + iota
            pv0 = lax.shift_right_logical(vvec0, 1)
            pv1 = lax.shift_right_logical(vvec1, 1)
            hv0 = lax.shift_left(lax.bitwise_and(vvec0, 1), 6)
            hv1 = lax.shift_left(lax.bitwise_and(vvec1, 1), 6)
            for dt in range(_D // 16):
                for k in range(16):
                    dvec = dt * 16 + lax.bitwise_and(k + iota, 15)
                    x0 = plsc.load_gather(slab, [dvec, vvec0])
                    x1 = plsc.load_gather(slab, [dvec, vvec1])
                    plsc.store_scatter(pair, [pv0, hv0 + dvec], x0)
                    plsc.store_scatter(pair, [pv1, hv1 + dvec], x1)
            return carry

        lax.fori_loop(0, _VB // 32, vbody, 0)

    def wout(b, pair, sem):
        pltpu.async_copy(
            pair, out_hbm.at[pl.ds(b * (_VB // 2), _VB // 2)], sem)

    def wait_o(pair, sem):
        pltpu.make_async_copy(
            pair, out_hbm.at[pl.ds(0, _VB // 2)], sem).wait()

    @pl.when(nblk > 0)
    def _():
        fire(wid, ia, isa)

    @pl.when(nblk > 1)
    def _():
        fire(wid + _NW, ib, isb)

    def body(i, carry):
        b0 = wid + 2 * i * _NW

        @pl.when(2 * i < nblk)
        def _():
            wait_i(ia, isa)

            @pl.when(2 * i > 1)
            def _():
                wait_o(pa, osa)

            xpose(ia, pa)
            wout(b0, pa, osa)

            @pl.when(2 * i + 2 < nblk)
            def _():
                fire(b0 + 2 * _NW, ia, isa)

        @pl.when(2 * i + 1 < nblk)
        def _():
            wait_i(ib, isb)

            @pl.when(2 * i > 0)
            def _():
                wait_o(pb, osb)

            xpose(ib, pb)
            wout(b0 + _NW, pb, osb)

            @pl.when(2 * i + 3 < nblk)
            def _():
                fire(b0 + 3 * _NW, ib, isb)

        return carry

    lax.fori_loop(0, (nblk + 1) // 2, body, 0)

    @pl.when(nblk > 0)
    def _():
        wait_o(pa, osa)

    @pl.when(nblk > 1)
    def _():
        wait_o(pb, osb)

    @pl.when(wid == _NW - 1)
    def _():
        pltpu.sync_copy(tt_hbm.at[:, pl.ds(_NVB * _VB, _VTAIL)], tin)

        def tbody(vt, carry):
            vvec0 = vt * 32 + iota
            vvec1 = vt * 32 + 16 + iota
            pv0 = lax.shift_right_logical(vvec0, 1)
            pv1 = lax.shift_right_logical(vvec1, 1)
            hv0 = lax.shift_left(lax.bitwise_and(vvec0, 1), 6)
            hv1 = lax.shift_left(lax.bitwise_and(vvec1, 1), 6)
            for dt in range(_D // 16):
                for k in range(16):
                    dvec = dt * 16 + lax.bitwise_and(k + iota, 15)
                    x0 = plsc.load_gather(tin, [dvec, vvec0])
                    x1 = plsc.load_gather(tin, [dvec, vvec1])
                    plsc.store_scatter(tout, [pv0, hv0 + dvec], x0)
                    plsc.store_scatter(tout, [pv1, hv1 + dvec], x1)
            return carry

        lax.fori_loop(0, _VTAIL // 32, tbody, 0)
        pltpu.sync_copy(
            tout, out_hbm.at[pl.ds(_NVB * _VB // 2, _VTAIL // 2)])


def kernel(x, table):
    x = x.astype(jnp.int32)
    xt = x.T
    tbl_pairs = _sc_repack(table.T)
    outp, mask_t = _sc_emb(xt, tbl_pairs)
    out_emb = jnp.transpose(outp, (0, 2, 1))
    mask = mask_t.T
    return (out_emb, mask)


# speedup vs baseline: 1.6326x; 1.2486x over previous
import functools

import jax
import jax.numpy as jnp
from jax import lax
from jax.experimental import pallas as pl
from jax.experimental.pallas import tpu as pltpu
from jax.experimental.pallas import tpu_sc as plsc

_VOCAB = 1000000
_D = 64
_B = 4096
_S = 200

try:
    _info = plsc.get_sparse_core_info()
    _NC, _NS = _info.num_cores, _info.num_subcores
except Exception:
    _NC, _NS = 2, 16
_NW = _NC * _NS
_BW = _B // _NW
_TROWS = _VOCAB // 2

_mesh = plsc.VectorSubcoreMesh(core_axis_name="c", subcore_axis_name="s")


@functools.partial(
    pl.kernel,
    mesh=_mesh,
    out_type=(
        jax.ShapeDtypeStruct((_S, _D, _B), jnp.float32),
        jax.ShapeDtypeStruct((_S, _B), jnp.float32),
    ),
    compiler_params=pltpu.CompilerParams(
        use_tc_tiling_on_sc=True, needs_layout_passes=False),
    scratch_types=[
        pltpu.VMEM((_S, _BW), jnp.int32),
        pltpu.VMEM((_S, _BW), jnp.float32),
        pltpu.VMEM((_BW,), jnp.int32),
        pltpu.VMEM((_BW,), jnp.int32),
        pltpu.VMEM((_BW,), jnp.int32),
        pltpu.VMEM((_BW,), jnp.int32),
        pltpu.VMEM((_BW, 128), jnp.float32),
        pltpu.VMEM((_BW, 128), jnp.float32),
        pltpu.VMEM((_D, _BW), jnp.float32),
        pltpu.VMEM((_D, _BW), jnp.float32),
        pltpu.SemaphoreType.DMA,
        pltpu.SemaphoreType.DMA,
        pltpu.SemaphoreType.DMA,
        pltpu.SemaphoreType.DMA,
        pltpu.SemaphoreType.DMA,
    ],
)
def _sc_emb(xt_hbm, tbl_hbm, out_hbm, mask_hbm, xt_all, mk_all, i2a, i2b,
            hfa, hfb, ra, rb, oa, ob, gsa, gsb, osa, osb, msem):
    wid = lax.axis_index("s") * _NC + lax.axis_index("c")
    b0 = wid * _BW
    pltpu.sync_copy(xt_hbm.at[:, pl.ds(b0, _BW)], xt_all)

    def build(s, i2, hf):
        for g in range(_BW // 16):
            tv = xt_all[s, pl.ds(g * 16, 16)]
            i2[pl.ds(g * 16, 16)] = lax.shift_right_logical(tv, 1)
            hf[pl.ds(g * 16, 16)] = lax.shift_left(
                lax.bitwise_and(tv, jnp.int32(1)), 6)

    def fire(i2, rows, sem):
        pltpu.async_copy(tbl_hbm.at[i2], rows, sem)

    def wait_g(i2, rows, sem):
        pltpu.make_async_copy(tbl_hbm.at[i2], rows, sem).wait()

    def compact(hf, rows, ov):
        iota = lax.iota(jnp.int32, 16)

        def gbody(g, carry):
            hv0 = hf[pl.ds(g * 32, 16)]
            hv1 = hf[pl.ds(g * 32 + 16, 16)]
            rowv0 = g * 32 + iota
            rowv1 = g * 32 + 16 + iota
            for hi in range(_D // 16):
                ovh = ov.at[pl.ds(hi * 16, 16)]
                hh0 = hv0 + hi * 16
                hh1 = hv1 + hi * 16
                for lo in range(16):
                    dlo = lax.bitwise_and(lo + iota, 15)
                    vals0 = plsc.load_gather(rows, [rowv0, hh0 + dlo])
                    vals1 = plsc.load_gather(rows, [rowv1, hh1 + dlo])
                    plsc.store_scatter(ovh, [dlo, rowv0], vals0)
                    plsc.store_scatter(ovh, [dlo, rowv1], vals1)
            return carry

        lax.fori_loop(0, _BW // 32, gbody, 0)

    def wout(s, ov, sem):
        pltpu.async_copy(ov, out_hbm.at[s, :, pl.ds(b0, _BW)], sem)

    def wait_o(ov, sem):
        pltpu.make_async_copy(ov, out_hbm.at[0, :, pl.ds(b0, _BW)], sem).wait()

    def mbody(s, carry):
        for g in range(_BW // 16):
            tv = xt_all[s, pl.ds(g * 16, 16)]
            mk_all[s, pl.ds(g * 16, 16)] = (tv != 0).astype(jnp.float32)
        return carry

    lax.fori_loop(0, _S, mbody, 0)
    pltpu.async_copy(mk_all, mask_hbm.at[:, pl.ds(b0, _BW)], msem)

    build(0, i2a, hfa)
    fire(i2a, ra, gsa)
    build(1, i2b, hfb)

    def body(t, carry):
        s0 = 2 * t
        fire(i2b, rb, gsb)
        wait_g(i2a, ra, gsa)

        @pl.when(t > 0)
        def _():
            wait_o(oa, osa)

        compact(hfa, ra, oa)
        wout(s0, oa, osa)

        @pl.when(t < _S // 2 - 1)
        def _():
            build(s0 + 2, i2a, hfa)
            fire(i2a, ra, gsa)

        wait_g(i2b, rb, gsb)

        @pl.when(t > 0)
        def _():
            wait_o(ob, osb)

        compact(hfb, rb, ob)
        wout(s0 + 1, ob, osb)

        @pl.when(t < _S // 2 - 1)
        def _():
            build(s0 + 3, i2b, hfb)

        return carry

    lax.fori_loop(0, _S // 2, body, 0)
    wait_o(oa, osa)
    wait_o(ob, osb)
    pltpu.make_async_copy(mk_all, mask_hbm.at[:, pl.ds(b0, _BW)], msem).wait()


_VB = 256
_NVB = _VOCAB // _VB
_VTAIL = _VOCAB - _NVB * _VB


@functools.partial(
    pl.kernel,
    mesh=_mesh,
    out_type=jax.ShapeDtypeStruct((_TROWS, 2 * _D), jnp.float32),
    compiler_params=pltpu.CompilerParams(
        use_tc_tiling_on_sc=True, needs_layout_passes=False),
    scratch_types=[
        pltpu.VMEM((_D, _VB), jnp.float32),
        pltpu.VMEM((_D, _VB), jnp.float32),
        pltpu.VMEM((_VB // 2, 2 * _D), jnp.float32),
        pltpu.VMEM((_VB // 2, 2 * _D), jnp.float32),
        pltpu.VMEM((_D, _VTAIL), jnp.float32),
        pltpu.VMEM((_VTAIL // 2, 2 * _D), jnp.float32),
        pltpu.SemaphoreType.DMA,
        pltpu.SemaphoreType.DMA,
        pltpu.SemaphoreType.DMA,
        pltpu.SemaphoreType.DMA,
    ],
)
def _sc_repack(tt_hbm, out_hbm, ia, ib, pa, pb, tin, tout,
               isa, isb, osa, osb):
    wid = lax.axis_index("s") * _NC + lax.axis_index("c")
    nblk = (_NVB - wid + _NW - 1) // _NW
    iota = lax.iota(jnp.int32, 16)

    def fire(b, slab, sem):
        pltpu.async_copy(tt_hbm.at[:, pl.ds(b * _VB, _VB)], slab, sem)

    def wait_i(slab, sem):
        pltpu.make_async_copy(tt_hbm.at[:, pl.ds(0, _VB)], slab, sem).wait()

    def xpose(slab, pair):
        def vbody(vt, carry):
            vvec0 = vt * 32 + iota
            vvec1 = vt * 32 + 16 + iota
            pv0 = lax.shift_right_logical(vvec0, 1)
            pv1 = lax.shift_right_logical(vvec1, 1)
            hv0 = lax.shift_left(lax.bitwise_and(vvec0, 1), 6)
            hv1 = lax.shift_left(lax.bitwise_and(vvec1, 1), 6)
            for dt in range(_D // 16):
                slabd = slab.at[pl.ds(dt * 16, 16)]
                hh0 = hv0 + dt * 16
                hh1 = hv1 + dt * 16
                for k in range(16):
                    dlo = lax.bitwise_and(k + iota, 15)
                    x0 = plsc.load_gather(slabd, [dlo, vvec0])
                    x1 = plsc.load_gather(slabd, [dlo, vvec1])
                    plsc.store_scatter(pair, [pv0, hh0 + dlo], x0)
                    plsc.store_scatter(pair, [pv1, hh1 + dlo], x1)
            return carry

        lax.fori_loop(0, _VB // 32, vbody, 0)

    def wout(b, pair, sem):
        pltpu.async_copy(
            pair, out_hbm.at[pl.ds(b * (_VB // 2), _VB // 2)], sem)

    def wait_o(pair, sem):
        pltpu.make_async_copy(
            pair, out_hbm.at[pl.ds(0, _VB // 2)], sem).wait()

    @pl.when(nblk > 0)
    def _():
        fire(wid, ia, isa)

    @pl.when(nblk > 1)
    def _():
        fire(wid + _NW, ib, isb)

    def body(i, carry):
        b0 = wid + 2 * i * _NW

        @pl.when(2 * i < nblk)
        def _():
            wait_i(ia, isa)

            @pl.when(2 * i > 1)
            def _():
                wait_o(pa, osa)

            xpose(ia, pa)
            wout(b0, pa, osa)

            @pl.when(2 * i + 2 < nblk)
            def _():
                fire(b0 + 2 * _NW, ia, isa)

        @pl.when(2 * i + 1 < nblk)
        def _():
            wait_i(ib, isb)

            @pl.when(2 * i > 0)
            def _():
                wait_o(pb, osb)

            xpose(ib, pb)
            wout(b0 + _NW, pb, osb)

            @pl.when(2 * i + 3 < nblk)
            def _():
                fire(b0 + 3 * _NW, ib, isb)

        return carry

    lax.fori_loop(0, (nblk + 1) // 2, body, 0)

    @pl.when(nblk > 0)
    def _():
        wait_o(pa, osa)

    @pl.when(nblk > 1)
    def _():
        wait_o(pb, osb)

    @pl.when(wid == _NW - 1)
    def _():
        pltpu.sync_copy(tt_hbm.at[:, pl.ds(_NVB * _VB, _VTAIL)], tin)

        def tbody(vt, carry):
            vvec0 = vt * 32 + iota
            vvec1 = vt * 32 + 16 + iota
            pv0 = lax.shift_right_logical(vvec0, 1)
            pv1 = lax.shift_right_logical(vvec1, 1)
            hv0 = lax.shift_left(lax.bitwise_and(vvec0, 1), 6)
            hv1 = lax.shift_left(lax.bitwise_and(vvec1, 1), 6)
            for dt in range(_D // 16):
                tind = tin.at[pl.ds(dt * 16, 16)]
                hh0 = hv0 + dt * 16
                hh1 = hv1 + dt * 16
                for k in range(16):
                    dlo = lax.bitwise_and(k + iota, 15)
                    x0 = plsc.load_gather(tind, [dlo, vvec0])
                    x1 = plsc.load_gather(tind, [dlo, vvec1])
                    plsc.store_scatter(tout, [pv0, hh0 + dlo], x0)
                    plsc.store_scatter(tout, [pv1, hh1 + dlo], x1)
            return carry

        lax.fori_loop(0, _VTAIL // 32, tbody, 0)
        pltpu.sync_copy(
            tout, out_hbm.at[pl.ds(_NVB * _VB // 2, _VTAIL // 2)])


def kernel(x, table):
    x = x.astype(jnp.int32)
    xt = x.T
    tbl_pairs = _sc_repack(table.T)
    outp, mask_t = _sc_emb(xt, tbl_pairs)
    out_emb = jnp.transpose(outp, (0, 2, 1))
    mask = mask_t.T
    return (out_emb, mask)
